# baseline (speedup 1.0000x reference)
import functools

import jax
import jax.numpy as jnp
from jax import lax
from jax.experimental import pallas as pl
from jax.experimental.pallas import tpu as pltpu
from jax.experimental.pallas import tpu_sc as plsc

N = 10000
NP = 10240
E = 320000
EPB = 200
BPT = 50
NT = 32
EPT = BPT * EPB
SPT = NP // 16


def _sc_mesh():
    return plsc.VectorSubcoreMesh(core_axis_name="c", subcore_axis_name="s")


_SC_PARAMS = pltpu.CompilerParams(use_tc_tiling_on_sc=False)


@functools.partial(
    pl.kernel,
    out_type=jax.ShapeDtypeStruct((NT, SPT), jnp.float32),
    mesh=_sc_mesh(),
    compiler_params=_SC_PARAMS,
    scratch_types=[
        pltpu.VMEM((EPT,), jnp.int32),
        pltpu.VMEM((EPB,), jnp.float32),
        pltpu.VMEM_SHARED((NP,), jnp.float32),
        pltpu.SemaphoreType.DMA,
    ],
)
def _deg_kernel(ei_hbm, ones_hbm, zeros_hbm, out_hbm, idx_d, ones_v, acc, sem):
    c = lax.axis_index("c")
    s = lax.axis_index("s")
    wid = c * 16 + s
    rsl = pl.ds(s * SPT, SPT)
    pltpu.sync_copy(zeros_hbm.at[rsl], acc.at[rsl])
    pltpu.sync_copy(ones_hbm, ones_v)
    pltpu.sync_copy(ei_hbm.at[1, pl.ds(wid * EPT, EPT)], idx_d)
    plsc.subcore_barrier()

    def body(j, carry):
        pltpu.async_copy(
            ones_v, acc.at[idx_d.at[pl.ds(j * EPB, EPB)]], sem, add=True)
        return carry

    lax.fori_loop(0, BPT, body, 0)

    def drain(j, carry):
        pltpu.make_async_copy(ones_hbm, ones_v, sem).wait()
        return carry

    lax.fori_loop(0, BPT, drain, 0)
    plsc.subcore_barrier()
    pltpu.sync_copy(acc.at[rsl], out_hbm.at[wid])


def _make_agg(D):
    @functools.partial(
        pl.kernel,
        out_type=jax.ShapeDtypeStruct((NT, SPT, D), jnp.float32),
        mesh=_sc_mesh(),
        compiler_params=_SC_PARAMS,
        scratch_types=[
            pltpu.VMEM((EPT,), jnp.int32),
            pltpu.VMEM((EPT,), jnp.int32),
            [pltpu.VMEM((EPB, D), jnp.float32)] * 2,
            pltpu.VMEM_SHARED((NP, D), jnp.float32),
            pltpu.SemaphoreType.DMA,
            pltpu.SemaphoreType.DMA,
        ],
    )
    def agg(ei_hbm, y_hbm, zeros_hbm, out_hbm,
            idx_s, idx_d, rows, acc, sem, semi):
        c = lax.axis_index("c")
        s = lax.axis_index("s")
        wid = c * 16 + s
        rsl = pl.ds(s * SPT, SPT)
        pltpu.sync_copy(zeros_hbm.at[rsl], acc.at[rsl])
        h1 = pltpu.async_copy(ei_hbm.at[0, pl.ds(wid * EPT, EPT)], idx_s, semi)
        h2 = pltpu.async_copy(ei_hbm.at[1, pl.ds(wid * EPT, EPT)], idx_d, semi)
        h1.wait()
        h2.wait()
        plsc.subcore_barrier()

        def drain2():
            for b in range(2):
                pltpu.make_async_copy(
                    y_hbm.at[pl.ds(0, EPB)], rows[b], semi).wait()

        def body(i, carry):
            j0 = i * 2 * EPB

            @pl.when(i > 0)
            def _():
                drain2()

            gs = [pltpu.async_copy(
                      y_hbm.at[idx_s.at[pl.ds(j0 + b * EPB, EPB)]],
                      rows[b], sem)
                  for b in range(2)]
            for b in range(2):
                gs[b].wait()
                pltpu.async_copy(
                    rows[b], acc.at[idx_d.at[pl.ds(j0 + b * EPB, EPB)]],
                    semi, add=True)
            return carry

        lax.fori_loop(0, BPT // 2, body, 0)
        drain2()
        plsc.subcore_barrier()
        pltpu.sync_copy(acc.at[rsl], out_hbm.at[wid])

    return agg


_agg64 = _make_agg(64)
_agg32 = _make_agg(32)


RB = 10000


def _dis(degp):
    return lax.rsqrt(degp[:, 0:1] + degp[:, 1:2] + 1.0)


def _mm_body(x_ref, w1_ref, xw_ref):
    xw_ref[...] = jnp.dot(
        x_ref[...], w1_ref[...], preferred_element_type=jnp.float32)


def _scale_body(degp_ref, xw_ref, y1_ref):
    y1_ref[...] = _dis(degp_ref[...]) * xw_ref[...]


def _mid_body(degp_ref, agg_ref, y1_ref, b1_ref, w2_ref, y2_ref):
    dis = _dis(degp_ref[...])
    a = agg_ref[0] + agg_ref[1] + y1_ref[...]
    h1 = jnp.maximum(dis * a + b1_ref[...], 0.0)
    y2_ref[...] = dis * jnp.dot(h1, w2_ref[...], preferred_element_type=jnp.float32)


def _post_body(degp_ref, agg_ref, y2_ref, b2_ref, wc_ref, bc_ref, out_ref):
    dis = _dis(degp_ref[...])
    a = agg_ref[0] + agg_ref[1] + y2_ref[...]
    h2 = jnp.maximum(dis * a + b2_ref[...], 0.0)
    out_ref[...] = (
        jnp.dot(h2, wc_ref[...], preferred_element_type=jnp.float32) + bc_ref[...])


def _row_spec(cols):
    return pl.BlockSpec((RB, cols), lambda i: (i, 0))


def _full_spec(shape):
    nd = len(shape)
    return pl.BlockSpec(shape, lambda i, _nd=nd: (0,) * _nd)


def _agg_spec(cols):
    return pl.BlockSpec((2, RB, cols), lambda i: (0, i, 0))


_GRID = N // RB

_mm = pl.pallas_call(
    _mm_body,
    grid=(_GRID,),
    in_specs=[_row_spec(128), _full_spec((128, 64))],
    out_specs=_row_spec(64),
    out_shape=jax.ShapeDtypeStruct((N, 64), jnp.float32),
)

_scale = pl.pallas_call(
    _scale_body,
    grid=(_GRID,),
    in_specs=[_row_spec(2), _row_spec(64)],
    out_specs=_row_spec(64),
    out_shape=jax.ShapeDtypeStruct((N, 64), jnp.float32),
)

_mid = pl.pallas_call(
    _mid_body,
    grid=(_GRID,),
    in_specs=[_row_spec(2), _agg_spec(64), _row_spec(64),
              _full_spec((1, 64)), _full_spec((64, 32))],
    out_specs=_row_spec(32),
    out_shape=jax.ShapeDtypeStruct((N, 32), jnp.float32),
)

_post = pl.pallas_call(
    _post_body,
    grid=(_GRID,),
    in_specs=[_row_spec(2), _agg_spec(32), _row_spec(32),
              _full_spec((1, 32)), _full_spec((32, 10)), _full_spec((1, 10))],
    out_specs=_row_spec(10),
    out_shape=jax.ShapeDtypeStruct((N, 10), jnp.float32),
)


def kernel(x, edge_index, W1, b1, W2, b2, Wc, bc):
    ei = edge_index.astype(jnp.int32)

    ones = jnp.ones((EPB,), jnp.float32)
    zeros1 = jnp.zeros((NP,), jnp.float32)
    zeros64 = jnp.zeros((NP, 64), jnp.float32)
    zeros32 = jnp.zeros((NP, 32), jnp.float32)

    degp = _deg_kernel(ei, ones, zeros1)
    degp = degp.reshape(2, NP).T
    xw1 = _mm(x, W1)
    y1 = _scale(degp, xw1)
    agg1 = _agg64(ei, y1, zeros64).reshape(2, NP, 64)
    y2 = _mid(degp, agg1, y1, b1.reshape(1, 64), W2)
    agg2 = _agg32(ei, y2, zeros32).reshape(2, NP, 32)
    return _post(degp, agg2, y2, b2.reshape(1, 32), Wc, bc.reshape(1, 10))


# speedup vs baseline: 1.0962x; 1.0962x over previous
import functools

import jax
import jax.numpy as jnp
from jax import lax
from jax.experimental import pallas as pl
from jax.experimental.pallas import tpu as pltpu
from jax.experimental.pallas import tpu_sc as plsc

N = 10000
NP = 10240
E = 320000
EPB = 200
BPT = 50
NT = 32
EPT = BPT * EPB
SPT = NP // 16


def _sc_mesh():
    return plsc.VectorSubcoreMesh(core_axis_name="c", subcore_axis_name="s")


_SC_PARAMS = pltpu.CompilerParams(use_tc_tiling_on_sc=False)


@functools.partial(
    pl.kernel,
    out_type=jax.ShapeDtypeStruct((NT, SPT), jnp.float32),
    mesh=_sc_mesh(),
    compiler_params=_SC_PARAMS,
    scratch_types=[
        pltpu.VMEM((EPT,), jnp.int32),
        pltpu.VMEM((EPB,), jnp.float32),
        pltpu.VMEM_SHARED((NP,), jnp.float32),
        pltpu.SemaphoreType.DMA,
    ],
)
def _deg_kernel(ei_hbm, ones_hbm, zeros_hbm, out_hbm, idx_d, ones_v, acc, sem):
    c = lax.axis_index("c")
    s = lax.axis_index("s")
    wid = c * 16 + s
    rsl = pl.ds(s * SPT, SPT)
    pltpu.sync_copy(zeros_hbm.at[rsl], acc.at[rsl])
    pltpu.sync_copy(ones_hbm, ones_v)
    pltpu.sync_copy(ei_hbm.at[1, pl.ds(wid * EPT, EPT)], idx_d)
    plsc.subcore_barrier()

    def body(j, carry):
        pltpu.async_copy(
            ones_v, acc.at[idx_d.at[pl.ds(j * EPB, EPB)]], sem, add=True)
        return carry

    lax.fori_loop(0, BPT, body, 0)

    def drain(j, carry):
        pltpu.make_async_copy(ones_hbm, ones_v, sem).wait()
        return carry

    lax.fori_loop(0, BPT, drain, 0)
    plsc.subcore_barrier()
    pltpu.sync_copy(acc.at[rsl], out_hbm.at[wid])


def _make_agg(D):
    @functools.partial(
        pl.kernel,
        out_type=jax.ShapeDtypeStruct((NT, SPT, D), jnp.float32),
        mesh=_sc_mesh(),
        compiler_params=_SC_PARAMS,
        scratch_types=[
            pltpu.VMEM((EPT,), jnp.int32),
            pltpu.VMEM((EPT,), jnp.int32),
            [pltpu.VMEM((EPB, D), jnp.float32)] * 5,
            pltpu.VMEM_SHARED((NP, D), jnp.float32),
            pltpu.SemaphoreType.DMA,
            pltpu.SemaphoreType.DMA,
        ],
    )
    def agg(ei_hbm, y_hbm, zeros_hbm, out_hbm,
            idx_s, idx_d, rows, acc, sem, semi):
        c = lax.axis_index("c")
        s = lax.axis_index("s")
        wid = c * 16 + s
        rsl = pl.ds(s * SPT, SPT)
        pltpu.sync_copy(zeros_hbm.at[rsl], acc.at[rsl])
        h1 = pltpu.async_copy(ei_hbm.at[0, pl.ds(wid * EPT, EPT)], idx_s, semi)
        h2 = pltpu.async_copy(ei_hbm.at[1, pl.ds(wid * EPT, EPT)], idx_d, semi)
        h1.wait()
        h2.wait()
        plsc.subcore_barrier()

        def drain5():
            for b in range(5):
                pltpu.make_async_copy(
                    y_hbm.at[pl.ds(0, EPB)], rows[b], semi).wait()

        def body(i, carry):
            j0 = i * 5 * EPB

            @pl.when(i > 0)
            def _():
                drain5()

            gs = [pltpu.async_copy(
                      y_hbm.at[idx_s.at[pl.ds(j0 + b * EPB, EPB)]],
                      rows[b], sem)
                  for b in range(5)]
            for b in range(5):
                gs[b].wait()
                pltpu.async_copy(
                    rows[b], acc.at[idx_d.at[pl.ds(j0 + b * EPB, EPB)]],
                    semi, add=True)
            return carry

        lax.fori_loop(0, BPT // 5, body, 0)
        drain5()
        plsc.subcore_barrier()
        pltpu.sync_copy(acc.at[rsl], out_hbm.at[wid])

    return agg


_agg64 = _make_agg(64)
_agg32 = _make_agg(32)


RB = 10000


def _dis(degp):
    return lax.rsqrt(degp[:, 0:1] + degp[:, 1:2] + 1.0)


def _mm_body(x_ref, w1_ref, xw_ref):
    xw_ref[...] = jnp.dot(
        x_ref[...], w1_ref[...], preferred_element_type=jnp.float32)


def _scale_body(degp_ref, xw_ref, y1_ref):
    y1_ref[...] = _dis(degp_ref[...]) * xw_ref[...]


def _mid_body(degp_ref, agg_ref, y1_ref, b1_ref, w2_ref, y2_ref):
    dis = _dis(degp_ref[...])
    a = agg_ref[0] + agg_ref[1] + y1_ref[...]
    h1 = jnp.maximum(dis * a + b1_ref[...], 0.0)
    y2_ref[...] = dis * jnp.dot(h1, w2_ref[...], preferred_element_type=jnp.float32)


def _post_body(degp_ref, agg_ref, y2_ref, b2_ref, wc_ref, bc_ref, out_ref):
    dis = _dis(degp_ref[...])
    a = agg_ref[0] + agg_ref[1] + y2_ref[...]
    h2 = jnp.maximum(dis * a + b2_ref[...], 0.0)
    out_ref[...] = (
        jnp.dot(h2, wc_ref[...], preferred_element_type=jnp.float32) + bc_ref[...])


def _row_spec(cols):
    return pl.BlockSpec((RB, cols), lambda i: (i, 0))


def _full_spec(shape):
    nd = len(shape)
    return pl.BlockSpec(shape, lambda i, _nd=nd: (0,) * _nd)


def _agg_spec(cols):
    return pl.BlockSpec((2, RB, cols), lambda i: (0, i, 0))


_GRID = N // RB

_mm = pl.pallas_call(
    _mm_body,
    grid=(_GRID,),
    in_specs=[_row_spec(128), _full_spec((128, 64))],
    out_specs=_row_spec(64),
    out_shape=jax.ShapeDtypeStruct((N, 64), jnp.float32),
)

_scale = pl.pallas_call(
    _scale_body,
    grid=(_GRID,),
    in_specs=[_row_spec(2), _row_spec(64)],
    out_specs=_row_spec(64),
    out_shape=jax.ShapeDtypeStruct((N, 64), jnp.float32),
)

_mid = pl.pallas_call(
    _mid_body,
    grid=(_GRID,),
    in_specs=[_row_spec(2), _agg_spec(64), _row_spec(64),
              _full_spec((1, 64)), _full_spec((64, 32))],
    out_specs=_row_spec(32),
    out_shape=jax.ShapeDtypeStruct((N, 32), jnp.float32),
)

_post = pl.pallas_call(
    _post_body,
    grid=(_GRID,),
    in_specs=[_row_spec(2), _agg_spec(32), _row_spec(32),
              _full_spec((1, 32)), _full_spec((32, 10)), _full_spec((1, 10))],
    out_specs=_row_spec(10),
    out_shape=jax.ShapeDtypeStruct((N, 10), jnp.float32),
)


def kernel(x, edge_index, W1, b1, W2, b2, Wc, bc):
    ei = edge_index.astype(jnp.int32)

    ones = jnp.ones((EPB,), jnp.float32)
    zeros1 = jnp.zeros((NP,), jnp.float32)
    zeros64 = jnp.zeros((NP, 64), jnp.float32)
    zeros32 = jnp.zeros((NP, 32), jnp.float32)

    degp = _deg_kernel(ei, ones, zeros1)
    degp = degp.reshape(2, NP).T
    xw1 = _mm(x, W1)
    y1 = _scale(degp, xw1)
    agg1 = _agg64(ei, y1, zeros64).reshape(2, NP, 64)
    y2 = _mid(degp, agg1, y1, b1.reshape(1, 64), W2)
    agg2 = _agg32(ei, y2, zeros32).reshape(2, NP, 32)
    return _post(degp, agg2, y2, b2.reshape(1, 32), Wc, bc.reshape(1, 10))


# speedup vs baseline: 1.1103x; 1.0128x over previous
import functools

import jax
import jax.numpy as jnp
from jax import lax
from jax.experimental import pallas as pl
from jax.experimental.pallas import tpu as pltpu
from jax.experimental.pallas import tpu_sc as plsc

N = 10000
NP = 10240
E = 320000
EPB = 200
BPT = 50
NT = 32
EPT = BPT * EPB
SPT = NP // 16


def _sc_mesh():
    return plsc.VectorSubcoreMesh(core_axis_name="c", subcore_axis_name="s")


_SC_PARAMS = pltpu.CompilerParams(use_tc_tiling_on_sc=False)


@functools.partial(
    pl.kernel,
    out_type=jax.ShapeDtypeStruct((NT, SPT), jnp.float32),
    mesh=_sc_mesh(),
    compiler_params=_SC_PARAMS,
    scratch_types=[
        pltpu.VMEM((EPT,), jnp.int32),
        pltpu.VMEM((EPB,), jnp.float32),
        pltpu.VMEM_SHARED((NP,), jnp.float32),
        pltpu.SemaphoreType.DMA,
    ],
)
def _deg_kernel(ei_hbm, ones_hbm, zeros_hbm, out_hbm, idx_d, ones_v, acc, sem):
    c = lax.axis_index("c")
    s = lax.axis_index("s")
    wid = c * 16 + s
    rsl = pl.ds(s * SPT, SPT)
    pltpu.sync_copy(zeros_hbm.at[rsl], acc.at[rsl])
    pltpu.sync_copy(ones_hbm, ones_v)
    pltpu.sync_copy(ei_hbm.at[1, pl.ds(wid * EPT, EPT)], idx_d)
    plsc.subcore_barrier()

    def body(j, carry):
        pltpu.async_copy(
            ones_v, acc.at[idx_d.at[pl.ds(j * EPB, EPB)]], sem, add=True)
        return carry

    lax.fori_loop(0, BPT, body, 0)

    def drain(j, carry):
        pltpu.make_async_copy(ones_hbm, ones_v, sem).wait()
        return carry

    lax.fori_loop(0, BPT, drain, 0)
    plsc.subcore_barrier()
    pltpu.sync_copy(acc.at[rsl], out_hbm.at[wid])


def _make_agg(D):
    @functools.partial(
        pl.kernel,
        out_type=jax.ShapeDtypeStruct((NT, SPT, D), jnp.float32),
        mesh=_sc_mesh(),
        compiler_params=_SC_PARAMS,
        scratch_types=[
            pltpu.VMEM((EPT,), jnp.int32),
            pltpu.VMEM((EPT,), jnp.int32),
            [pltpu.VMEM((EPB, D), jnp.float32)] * 5,
            pltpu.VMEM_SHARED((NP, D), jnp.float32),
            pltpu.SemaphoreType.DMA,
            pltpu.SemaphoreType.DMA,
        ],
    )
    def agg(ei_hbm, y_hbm, zeros_hbm, out_hbm,
            idx_s, idx_d, rows, acc, sem, semi):
        c = lax.axis_index("c")
        s = lax.axis_index("s")
        wid = c * 16 + s
        rsl = pl.ds(s * SPT, SPT)
        pltpu.sync_copy(zeros_hbm.at[rsl], acc.at[rsl])
        h1 = pltpu.async_copy(ei_hbm.at[0, pl.ds(wid * EPT, EPT)], idx_s, semi)
        h2 = pltpu.async_copy(ei_hbm.at[1, pl.ds(wid * EPT, EPT)], idx_d, semi)
        h1.wait()
        h2.wait()
        plsc.subcore_barrier()

        def drain5():
            for b in range(5):
                pltpu.make_async_copy(
                    y_hbm.at[pl.ds(0, EPB)], rows[b], semi).wait()

        def body(i, carry):
            j0 = i * 5 * EPB

            @pl.when(i > 0)
            def _():
                drain5()

            gs = [pltpu.async_copy(
                      y_hbm.at[idx_s.at[pl.ds(j0 + b * EPB, EPB)]],
                      rows[b], sem)
                  for b in range(5)]
            for b in range(5):
                gs[b].wait()
                pltpu.async_copy(
                    rows[b], acc.at[idx_d.at[pl.ds(j0 + b * EPB, EPB)]],
                    semi, add=True)
            return carry

        lax.fori_loop(0, BPT // 5, body, 0)
        drain5()
        plsc.subcore_barrier()
        pltpu.sync_copy(acc.at[rsl], out_hbm.at[wid])

    return agg


_agg64 = _make_agg(64)
_agg32 = _make_agg(32)


RB = 10000


def _dis(degp):
    dpt = jnp.transpose(degp)[:RB]
    return lax.rsqrt(dpt[:, 0:1] + dpt[:, 1:2] + 1.0)


def _mm_body(x_ref, w1_ref, xw_ref):
    xw_ref[...] = jnp.dot(
        x_ref[...], w1_ref[...], preferred_element_type=jnp.float32)


def _scale_body(degp_ref, xw_ref, y1_ref):
    y1_ref[...] = _dis(degp_ref[...]) * xw_ref[...]


def _mid_body(degp_ref, agg_ref, y1_ref, b1_ref, w2_ref, y2_ref):
    dis = _dis(degp_ref[...])
    a = agg_ref[0] + agg_ref[1] + y1_ref[...]
    h1 = jnp.maximum(dis * a + b1_ref[...], 0.0)
    y2_ref[...] = dis * jnp.dot(h1, w2_ref[...], preferred_element_type=jnp.float32)


def _post_body(degp_ref, agg_ref, y2_ref, b2_ref, wc_ref, bc_ref, out_ref):
    dis = _dis(degp_ref[...])
    a = agg_ref[0] + agg_ref[1] + y2_ref[...]
    h2 = jnp.maximum(dis * a + b2_ref[...], 0.0)
    out_ref[...] = (
        jnp.dot(h2, wc_ref[...], preferred_element_type=jnp.float32) + bc_ref[...])


def _row_spec(cols):
    return pl.BlockSpec((RB, cols), lambda i: (i, 0))


_deg_spec = pl.BlockSpec((2, NP), lambda i: (0, 0))


def _full_spec(shape):
    nd = len(shape)
    return pl.BlockSpec(shape, lambda i, _nd=nd: (0,) * _nd)


def _agg_spec(cols):
    return pl.BlockSpec((2, RB, cols), lambda i: (0, i, 0))


_GRID = N // RB

_mm = pl.pallas_call(
    _mm_body,
    grid=(_GRID,),
    in_specs=[_row_spec(128), _full_spec((128, 64))],
    out_specs=_row_spec(64),
    out_shape=jax.ShapeDtypeStruct((N, 64), jnp.float32),
)

_scale = pl.pallas_call(
    _scale_body,
    grid=(_GRID,),
    in_specs=[_deg_spec, _row_spec(64)],
    out_specs=_row_spec(64),
    out_shape=jax.ShapeDtypeStruct((N, 64), jnp.float32),
)

_mid = pl.pallas_call(
    _mid_body,
    grid=(_GRID,),
    in_specs=[_deg_spec, _agg_spec(64), _row_spec(64),
              _full_spec((1, 64)), _full_spec((64, 32))],
    out_specs=_row_spec(32),
    out_shape=jax.ShapeDtypeStruct((N, 32), jnp.float32),
)

_post = pl.pallas_call(
    _post_body,
    grid=(_GRID,),
    in_specs=[_deg_spec, _agg_spec(32), _row_spec(32),
              _full_spec((1, 32)), _full_spec((32, 10)), _full_spec((1, 10))],
    out_specs=_row_spec(10),
    out_shape=jax.ShapeDtypeStruct((N, 10), jnp.float32),
)


def kernel(x, edge_index, W1, b1, W2, b2, Wc, bc):
    ei = edge_index.astype(jnp.int32)

    ones = jnp.ones((EPB,), jnp.float32)
    zeros1 = jnp.zeros((NP,), jnp.float32)
    zeros64 = jnp.zeros((NP, 64), jnp.float32)
    zeros32 = jnp.zeros((NP, 32), jnp.float32)

    degp = _deg_kernel(ei, ones, zeros1).reshape(2, NP)
    xw1 = _mm(x, W1)
    y1 = _scale(degp, xw1)
    agg1 = _agg64(ei, y1, zeros64).reshape(2, NP, 64)
    y2 = _mid(degp, agg1, y1, b1.reshape(1, 64), W2)
    agg2 = _agg32(ei, y2, zeros32).reshape(2, NP, 32)
    return _post(degp, agg2, y2, b2.reshape(1, 32), Wc, bc.reshape(1, 10))
